# baseline (speedup 1.0000x reference)
import functools

import jax
import jax.numpy as jnp
from jax import lax
from jax.experimental import pallas as pl
from jax.experimental.pallas import tpu as pltpu
from jax.experimental.pallas import tpu_sc as plsc

B = 16384
D = 128
NC = 2
NS = 16
W0 = 576
W1 = B // NS - W0


@functools.partial(
    pl.kernel,
    mesh=plsc.VectorSubcoreMesh(core_axis_name="c", subcore_axis_name="s"),
    out_type=jax.ShapeDtypeStruct((B, D), jnp.float32),
    scratch_types=[
        pltpu.VMEM((W0,), jnp.int32),
        pltpu.VMEM((W0, D), jnp.float32),
        pltpu.SemaphoreType.DMA,
    ],
)
def _pe_gather(idx_hbm, table_hbm, out_hbm, idx_v, rows_v, sem):
    c = lax.axis_index("c")
    s = lax.axis_index("s")

    @pl.when(c == 0)
    def _():
        base = s * W0
        pltpu.sync_copy(idx_hbm.at[pl.ds(base, W0)], idx_v)
        pltpu.async_copy(table_hbm.at[idx_v], rows_v, sem).wait()
        pltpu.sync_copy(rows_v, out_hbm.at[pl.ds(base, W0)])

    @pl.when(c == 1)
    def _():
        base = NS * W0 + s * W1
        pltpu.sync_copy(
            idx_hbm.at[pl.ds(base, W1)], idx_v.at[pl.ds(0, W1)]
        )
        pltpu.async_copy(
            table_hbm.at[idx_v.at[pl.ds(0, W1)]],
            rows_v.at[pl.ds(0, W1)],
            sem,
        ).wait()
        pltpu.sync_copy(rows_v.at[pl.ds(0, W1)], out_hbm.at[pl.ds(base, W1)])


def kernel(t, pos_encoding):
    idx = t.astype(jnp.int32).reshape(B)
    return _pe_gather(idx, pos_encoding)


# speedup vs baseline: 1.0245x; 1.0245x over previous
import functools

import jax
import jax.numpy as jnp
from jax import lax
from jax.experimental import pallas as pl
from jax.experimental.pallas import tpu as pltpu
from jax.experimental.pallas import tpu_sc as plsc

B = 16384
D = 128
NC = 2
NS = 16
W0 = 448
W1 = B // NS - W0
WMAX = max(W0, W1)


@functools.partial(
    pl.kernel,
    mesh=plsc.VectorSubcoreMesh(core_axis_name="c", subcore_axis_name="s"),
    out_type=jax.ShapeDtypeStruct((B, D), jnp.float32),
    scratch_types=[
        pltpu.VMEM((WMAX,), jnp.int32),
        pltpu.VMEM((WMAX, D), jnp.float32),
        pltpu.SemaphoreType.DMA,
    ],
)
def _pe_gather(idx_hbm, table_hbm, out_hbm, idx_v, rows_v, sem):
    c = lax.axis_index("c")
    s = lax.axis_index("s")

    @pl.when(c == 0)
    def _():
        base = s * W0
        pltpu.sync_copy(
            idx_hbm.at[pl.ds(base, W0)], idx_v.at[pl.ds(0, W0)]
        )
        pltpu.async_copy(
            table_hbm.at[idx_v.at[pl.ds(0, W0)]],
            rows_v.at[pl.ds(0, W0)],
            sem,
        ).wait()
        pltpu.sync_copy(rows_v.at[pl.ds(0, W0)], out_hbm.at[pl.ds(base, W0)])

    @pl.when(c == 1)
    def _():
        base = NS * W0 + s * W1
        pltpu.sync_copy(
            idx_hbm.at[pl.ds(base, W1)], idx_v.at[pl.ds(0, W1)]
        )
        pltpu.async_copy(
            table_hbm.at[idx_v.at[pl.ds(0, W1)]],
            rows_v.at[pl.ds(0, W1)],
            sem,
        ).wait()
        pltpu.sync_copy(rows_v.at[pl.ds(0, W1)], out_hbm.at[pl.ds(base, W1)])


def kernel(t, pos_encoding):
    idx = t.astype(jnp.int32).reshape(B)
    return _pe_gather(idx, pos_encoding)


# speedup vs baseline: 1.0299x; 1.0052x over previous
import functools

import jax
import jax.numpy as jnp
from jax import lax
from jax.experimental import pallas as pl
from jax.experimental.pallas import tpu as pltpu
from jax.experimental.pallas import tpu_sc as plsc

B = 16384
D = 128
NC = 2
NS = 16
NW = NC * NS
B_PER_W = B // NW


@functools.partial(
    pl.kernel,
    mesh=plsc.VectorSubcoreMesh(core_axis_name="c", subcore_axis_name="s"),
    out_type=jax.ShapeDtypeStruct((B, D), jnp.float32),
    scratch_types=[
        pltpu.VMEM((B_PER_W,), jnp.int32),
        pltpu.VMEM((B_PER_W, D), jnp.float32),
        pltpu.SemaphoreType.DMA,
    ],
)
def _pe_gather(idx_hbm, table_hbm, out_hbm, idx_v, rows_v, sem):
    wid = lax.axis_index("s") * NC + lax.axis_index("c")
    base = wid * B_PER_W
    pltpu.sync_copy(idx_hbm.at[pl.ds(base, B_PER_W)], idx_v)
    pltpu.async_copy(table_hbm.at[idx_v], rows_v, sem).wait()
    pltpu.sync_copy(rows_v, out_hbm.at[pl.ds(base, B_PER_W)])


def kernel(t, pos_encoding):
    idx = t.astype(jnp.int32).reshape(B)
    return _pe_gather(idx, pos_encoding)
